# Initial kernel scaffold; baseline (speedup 1.0000x reference)
import jax
import jax.numpy as jnp
from jax.experimental import pallas as pl


def kernel(x, edge_index, W_pre, b_pre, W1_0, b1_0, W2_0, b2_0, W1_1, b1_1, W2_1, b2_1, W1_2, b1_2, W2_2, b2_2, gamma_0, beta_0, gamma_1, beta_1, Wp1, bp1, Wp2, bp2):
    raise NotImplementedError("write your pallas kernel here")


# speedup vs baseline: 2.5541x; 2.5541x over previous
import functools

import jax
import jax.numpy as jnp
from jax import lax
from jax.experimental import pallas as pl
from jax.experimental.pallas import tpu as pltpu
from jax.experimental.pallas import tpu_sc as plsc

N = 10000
E = 320000
F_IN = 128
H = 128
OUT = 8

NC = 2
NS = 16
NW = NC * NS
CHUNK = 128
NCHUNK = 80
EPT = NCHUNK * CHUNK
RPT = 632
ACC_ROWS = NS * RPT

ROWB = 2000
GRID = N // ROWB


def _seg_sum_body(h_hbm, src_hbm, dst_hbm, out_hbm,
                  src_v, dst_v, rows_v, acc, sem):
    c = lax.axis_index("c")
    s = lax.axis_index("s")
    wid = c * NS + s

    def zstore(i, carry):
        rows_v[i // 8, pl.ds((i % 8) * 16, 16)] = jnp.zeros((16,), jnp.float32)
        return carry
    lax.fori_loop(0, CHUNK * 8, zstore, 0)
    for t in range(4):
        pltpu.sync_copy(rows_v, acc.at[pl.ds(s * RPT + t * CHUNK, CHUNK)])
    pltpu.sync_copy(rows_v.at[pl.ds(0, RPT - 4 * CHUNK)],
                    acc.at[pl.ds(s * RPT + 4 * CHUNK, RPT - 4 * CHUNK)])

    pltpu.sync_copy(src_hbm.at[wid], src_v)
    pltpu.sync_copy(dst_hbm.at[wid], dst_v)
    plsc.subcore_barrier()

    def step(j, carry):
        pltpu.async_copy(h_hbm.at[src_v.at[j]], rows_v, sem).wait()
        pltpu.sync_copy(rows_v, acc.at[dst_v.at[j]], add=True)
        return carry
    lax.fori_loop(0, NCHUNK, step, 0)

    plsc.subcore_barrier()
    pltpu.sync_copy(acc.at[pl.ds(s * RPT, RPT)],
                    out_hbm.at[c, pl.ds(s * RPT, RPT)])


@functools.partial(
    pl.kernel,
    out_type=jax.ShapeDtypeStruct((NC, ACC_ROWS, H), jnp.float32),
    mesh=plsc.VectorSubcoreMesh(core_axis_name="c", subcore_axis_name="s"),
    scratch_types=[
        pltpu.VMEM((NCHUNK, CHUNK), jnp.int32),
        pltpu.VMEM((NCHUNK, CHUNK), jnp.int32),
        pltpu.VMEM((CHUNK, H), jnp.float32),
        pltpu.VMEM_SHARED((ACC_ROWS, H), jnp.float32),
        pltpu.SemaphoreType.DMA,
    ],
)
def _seg_sum(h_hbm, src_hbm, dst_hbm, out_hbm, src_v, dst_v, rows_v,
             acc, sem):
    _seg_sum_body(h_hbm, src_hbm, dst_hbm, out_hbm,
                  src_v, dst_v, rows_v, acc, sem)


def _leaky(z):
    return jnp.where(z >= 0, z, 0.01 * z)


def _pre_body(x_ref, w_ref, b_ref, o_ref):
    o_ref[...] = lax.dot_general(
        x_ref[...], w_ref[...], (((0,), (0,)), ((), ())),
        preferred_element_type=jnp.float32) + b_ref[...]


def _pre(x, W, b):
    return pl.pallas_call(
        _pre_body,
        out_shape=jax.ShapeDtypeStruct((N, H), jnp.float32),
    )(x, W, b)


def _conv_stats_body(h_ref, a0_ref, a1_ref, w1_ref, b1_ref, w2_ref, b2_ref,
                     u_ref, st_ref, acc_ref):
    i = pl.program_id(0)
    z = h_ref[...] + a0_ref[0] + a1_ref[0]
    t = _leaky(jnp.dot(z, w1_ref[...], preferred_element_type=jnp.float32)
               + b1_ref[...])
    u = jnp.dot(t, w2_ref[...], preferred_element_type=jnp.float32) + b2_ref[...]
    u_ref[...] = u
    blk = jnp.concatenate(
        [jnp.sum(u, axis=0, keepdims=True),
         jnp.sum(u * u, axis=0, keepdims=True),
         jnp.zeros((6, H), jnp.float32)], axis=0)

    @pl.when(i == 0)
    def _():
        acc_ref[...] = blk

    @pl.when(i > 0)
    def _():
        acc_ref[...] = acc_ref[...] + blk

    @pl.when(i == GRID - 1)
    def _():
        st_ref[...] = acc_ref[...]


def _conv_stats(h, parts, W1, b1, W2, b2):
    full = lambda shape: pl.BlockSpec(shape, lambda i: (0, 0))
    row = pl.BlockSpec((ROWB, H), lambda i: (i, 0))
    p0 = pl.BlockSpec((1, ROWB, H), lambda i: (0, i, 0))
    p1 = pl.BlockSpec((1, ROWB, H), lambda i: (1, i, 0))
    return pl.pallas_call(
        _conv_stats_body,
        grid=(GRID,),
        in_specs=[row, p0, p1, full((H, H)), full((1, H)), full((H, H)),
                  full((1, H))],
        out_specs=[row, full((8, H))],
        out_shape=[jax.ShapeDtypeStruct((N, H), jnp.float32),
                   jax.ShapeDtypeStruct((8, H), jnp.float32)],
        scratch_shapes=[pltpu.VMEM((8, H), jnp.float32)],
    )(h, parts, parts, W1, b1, W2, b2)


def _bn_body(u_ref, st_ref, g_ref, b_ref, o_ref):
    m = st_ref[0:1, :] / N
    v = st_ref[1:2, :] / N - m * m
    inv = lax.rsqrt(v + 1e-5)
    o_ref[...] = (u_ref[...] - m) * inv * g_ref[...] + b_ref[...]


def _bn(u, st, g, b):
    full = lambda shape: pl.BlockSpec(shape, lambda i: (0, 0))
    row = pl.BlockSpec((ROWB, H), lambda i: (i, 0))
    return pl.pallas_call(
        _bn_body,
        grid=(GRID,),
        in_specs=[row, full((8, H)), full((1, H)), full((1, H))],
        out_specs=row,
        out_shape=jax.ShapeDtypeStruct((N, H), jnp.float32),
    )(u, st, g, b)


def _last_body(h_ref, a0_ref, a1_ref, w1_ref, b1_ref, w2_ref, b2_ref,
               wp1_ref, bp1_ref, wp2_ref, bp2_ref, o_ref):
    z = h_ref[...] + a0_ref[0] + a1_ref[0]
    t = _leaky(jnp.dot(z, w1_ref[...], preferred_element_type=jnp.float32)
               + b1_ref[...])
    u = jnp.dot(t, w2_ref[...], preferred_element_type=jnp.float32) + b2_ref[...]
    p = _leaky(jnp.dot(u, wp1_ref[...], preferred_element_type=jnp.float32)
               + bp1_ref[...])
    o_ref[...] = jnp.dot(p, wp2_ref[...],
                         preferred_element_type=jnp.float32) + bp2_ref[...]


def _last(h, parts, W1, b1, W2, b2, Wp1, bp1, Wp2, bp2):
    full = lambda shape: pl.BlockSpec(shape, lambda i: (0, 0))
    row = pl.BlockSpec((ROWB, H), lambda i: (i, 0))
    p0 = pl.BlockSpec((1, ROWB, H), lambda i: (0, i, 0))
    p1 = pl.BlockSpec((1, ROWB, H), lambda i: (1, i, 0))
    return pl.pallas_call(
        _last_body,
        grid=(GRID,),
        in_specs=[row, p0, p1, full((H, H)), full((1, H)), full((H, H)),
                  full((1, H)), full((H, H)), full((1, H)), full((H, OUT)),
                  full((1, OUT))],
        out_specs=pl.BlockSpec((ROWB, OUT), lambda i: (i, 0)),
        out_shape=jax.ShapeDtypeStruct((N, OUT), jnp.float32),
    )(h, parts, parts, W1, b1, W2, b2, Wp1, bp1, Wp2, bp2)


def kernel(x, edge_index, W_pre, b_pre, W1_0, b1_0, W2_0, b2_0, W1_1, b1_1,
           W2_1, b2_1, W1_2, b1_2, W2_2, b2_2, gamma_0, beta_0, gamma_1,
           beta_1, Wp1, bp1, Wp2, bp2):
    pad = NW * EPT - E
    src = jnp.concatenate([edge_index[0], jnp.zeros((pad,), jnp.int32)])
    dst = jnp.concatenate([edge_index[1], jnp.full((pad,), N, jnp.int32)])
    srcp = src.reshape(NW, NCHUNK, CHUNK)
    dstp = dst.reshape(NW, NCHUNK, CHUNK)

    h = _pre(x, W_pre, b_pre.reshape(1, H))

    layers = [(W1_0, b1_0, W2_0, b2_0), (W1_1, b1_1, W2_1, b2_1),
              (W1_2, b1_2, W2_2, b2_2)]
    bns = [(gamma_0, beta_0), (gamma_1, beta_1)]

    for l, (W1, b1, W2, b2) in enumerate(layers):
        parts = _seg_sum(h, srcp, dstp)
        if l < 2:
            u, st = _conv_stats(h, parts, W1, b1.reshape(1, H), W2,
                                b2.reshape(1, H))
            g, bb = bns[l]
            h = _bn(u, st, g.reshape(1, H), bb.reshape(1, H))
        else:
            out = _last(h, parts, W1, b1.reshape(1, H), W2, b2.reshape(1, H),
                        Wp1, bp1.reshape(1, H), Wp2, bp2.reshape(1, OUT))
    return out.reshape(1, -1)


# speedup vs baseline: 2.8136x; 1.1016x over previous
import functools

import jax
import jax.numpy as jnp
from jax import lax
from jax.experimental import pallas as pl
from jax.experimental.pallas import tpu as pltpu
from jax.experimental.pallas import tpu_sc as plsc

N = 10000
E = 320000
F_IN = 128
H = 128
OUT = 8

NC = 2
NS = 16
NW = NC * NS
CHUNK = 128
NCHUNK = 80
NPAIR = NCHUNK // 2
EPT = NCHUNK * CHUNK
RPT = 632
ACC_ROWS = NS * RPT

ROWB = 2000
GRID = N // ROWB


def _seg_sum_body(h_hbm, idx_hbm, out_hbm,
                  idx_v, rows_v, acc, sem0, sem1, isem):
    c = lax.axis_index("c")
    s = lax.axis_index("s")
    wid = c * NS + s

    def zstore(i, carry):
        rows_v[0, i // 8, pl.ds((i % 8) * 16, 16)] = jnp.zeros((16,),
                                                               jnp.float32)
        return carry
    lax.fori_loop(0, CHUNK * 8, zstore, 0)
    for t in range(4):
        pltpu.sync_copy(rows_v.at[0],
                        acc.at[pl.ds(s * RPT + t * CHUNK, CHUNK)])
    pltpu.sync_copy(rows_v.at[0, pl.ds(0, RPT - 4 * CHUNK)],
                    acc.at[pl.ds(s * RPT + 4 * CHUNK, RPT - 4 * CHUNK)])
    plsc.subcore_barrier()

    def gather(buf, idx_ref, sem):
        return pltpu.async_copy(h_hbm.at[idx_ref], rows_v.at[buf], sem)

    def scatter(buf, idx_ref):
        pltpu.sync_copy(rows_v.at[buf], acc.at[idx_ref], add=True)

    pltpu.sync_copy(idx_hbm.at[wid, 0], idx_v.at[0])
    gather(0, idx_v.at[0, 0, 0], sem0)

    def pair(k, carry):
        p = lax.rem(k, 2)
        q = 1 - p
        ildma = pltpu.async_copy(idx_hbm.at[wid, k + 1], idx_v.at[q], isem)
        g1 = gather(1, idx_v.at[p, 1, 0], sem1)
        pltpu.make_async_copy(h_hbm.at[idx_v.at[p, 0, 0]], rows_v.at[0],
                              sem0).wait()
        scatter(0, idx_v.at[p, 0, 1])
        ildma.wait()
        gather(0, idx_v.at[q, 0, 0], sem0)
        g1.wait()
        scatter(1, idx_v.at[p, 1, 1])
        return carry
    lax.fori_loop(0, NPAIR - 1, pair, 0)

    g1 = gather(1, idx_v.at[1, 1, 0], sem1)
    pltpu.make_async_copy(h_hbm.at[idx_v.at[1, 0, 0]], rows_v.at[0],
                          sem0).wait()
    scatter(0, idx_v.at[1, 0, 1])
    g1.wait()
    scatter(1, idx_v.at[1, 1, 1])

    plsc.subcore_barrier()
    pltpu.sync_copy(acc.at[pl.ds(s * RPT, RPT)],
                    out_hbm.at[c, pl.ds(s * RPT, RPT)])


@functools.partial(
    pl.kernel,
    out_type=jax.ShapeDtypeStruct((NC, ACC_ROWS, H), jnp.float32),
    mesh=plsc.VectorSubcoreMesh(core_axis_name="c", subcore_axis_name="s"),
    scratch_types=[
        pltpu.VMEM((2, 2, 2, CHUNK), jnp.int32),
        pltpu.VMEM((2, CHUNK, H), jnp.float32),
        pltpu.VMEM_SHARED((ACC_ROWS, H), jnp.float32),
        pltpu.SemaphoreType.DMA,
        pltpu.SemaphoreType.DMA,
        pltpu.SemaphoreType.DMA,
    ],
)
def _seg_sum(h_hbm, idx_hbm, out_hbm, idx_v, rows_v, acc, sem0, sem1, isem):
    _seg_sum_body(h_hbm, idx_hbm, out_hbm,
                  idx_v, rows_v, acc, sem0, sem1, isem)


def _leaky(z):
    return jnp.where(z >= 0, z, 0.01 * z)


def _pre_body(x_ref, w_ref, b_ref, o_ref):
    o_ref[...] = lax.dot_general(
        x_ref[...], w_ref[...], (((0,), (0,)), ((), ())),
        preferred_element_type=jnp.float32) + b_ref[...]


def _pre(x, W, b):
    return pl.pallas_call(
        _pre_body,
        out_shape=jax.ShapeDtypeStruct((N, H), jnp.float32),
    )(x, W, b)


def _conv_stats_body(h_ref, a0_ref, a1_ref, w1_ref, b1_ref, w2_ref, b2_ref,
                     u_ref, st_ref, acc_ref):
    i = pl.program_id(0)
    z = h_ref[...] + a0_ref[0] + a1_ref[0]
    t = _leaky(jnp.dot(z, w1_ref[...], preferred_element_type=jnp.float32)
               + b1_ref[...])
    u = jnp.dot(t, w2_ref[...], preferred_element_type=jnp.float32) + b2_ref[...]
    u_ref[...] = u
    blk = jnp.concatenate(
        [jnp.sum(u, axis=0, keepdims=True),
         jnp.sum(u * u, axis=0, keepdims=True),
         jnp.zeros((6, H), jnp.float32)], axis=0)

    @pl.when(i == 0)
    def _():
        acc_ref[...] = blk

    @pl.when(i > 0)
    def _():
        acc_ref[...] = acc_ref[...] + blk

    @pl.when(i == GRID - 1)
    def _():
        st_ref[...] = acc_ref[...]


def _conv_stats(h, parts, W1, b1, W2, b2):
    full = lambda shape: pl.BlockSpec(shape, lambda i: (0, 0))
    row = pl.BlockSpec((ROWB, H), lambda i: (i, 0))
    p0 = pl.BlockSpec((1, ROWB, H), lambda i: (0, i, 0))
    p1 = pl.BlockSpec((1, ROWB, H), lambda i: (1, i, 0))
    return pl.pallas_call(
        _conv_stats_body,
        grid=(GRID,),
        in_specs=[row, p0, p1, full((H, H)), full((1, H)), full((H, H)),
                  full((1, H))],
        out_specs=[row, full((8, H))],
        out_shape=[jax.ShapeDtypeStruct((N, H), jnp.float32),
                   jax.ShapeDtypeStruct((8, H), jnp.float32)],
        scratch_shapes=[pltpu.VMEM((8, H), jnp.float32)],
    )(h, parts, parts, W1, b1, W2, b2)


def _bn_body(u_ref, st_ref, g_ref, b_ref, o_ref):
    m = st_ref[0:1, :] / N
    v = st_ref[1:2, :] / N - m * m
    inv = lax.rsqrt(v + 1e-5)
    o_ref[...] = (u_ref[...] - m) * inv * g_ref[...] + b_ref[...]


def _bn(u, st, g, b):
    full = lambda shape: pl.BlockSpec(shape, lambda i: (0, 0))
    row = pl.BlockSpec((ROWB, H), lambda i: (i, 0))
    return pl.pallas_call(
        _bn_body,
        grid=(GRID,),
        in_specs=[row, full((8, H)), full((1, H)), full((1, H))],
        out_specs=row,
        out_shape=jax.ShapeDtypeStruct((N, H), jnp.float32),
    )(u, st, g, b)


def _last_body(h_ref, a0_ref, a1_ref, w1_ref, b1_ref, w2_ref, b2_ref,
               wp1_ref, bp1_ref, wp2_ref, bp2_ref, o_ref):
    z = h_ref[...] + a0_ref[0] + a1_ref[0]
    t = _leaky(jnp.dot(z, w1_ref[...], preferred_element_type=jnp.float32)
               + b1_ref[...])
    u = jnp.dot(t, w2_ref[...], preferred_element_type=jnp.float32) + b2_ref[...]
    p = _leaky(jnp.dot(u, wp1_ref[...], preferred_element_type=jnp.float32)
               + bp1_ref[...])
    o_ref[...] = jnp.dot(p, wp2_ref[...],
                         preferred_element_type=jnp.float32) + bp2_ref[...]


def _last(h, parts, W1, b1, W2, b2, Wp1, bp1, Wp2, bp2):
    full = lambda shape: pl.BlockSpec(shape, lambda i: (0, 0))
    row = pl.BlockSpec((ROWB, H), lambda i: (i, 0))
    p0 = pl.BlockSpec((1, ROWB, H), lambda i: (0, i, 0))
    p1 = pl.BlockSpec((1, ROWB, H), lambda i: (1, i, 0))
    return pl.pallas_call(
        _last_body,
        grid=(GRID,),
        in_specs=[row, p0, p1, full((H, H)), full((1, H)), full((H, H)),
                  full((1, H)), full((H, H)), full((1, H)), full((H, OUT)),
                  full((1, OUT))],
        out_specs=pl.BlockSpec((ROWB, OUT), lambda i: (i, 0)),
        out_shape=jax.ShapeDtypeStruct((N, OUT), jnp.float32),
    )(h, parts, parts, W1, b1, W2, b2, Wp1, bp1, Wp2, bp2)


def kernel(x, edge_index, W_pre, b_pre, W1_0, b1_0, W2_0, b2_0, W1_1, b1_1,
           W2_1, b2_1, W1_2, b1_2, W2_2, b2_2, gamma_0, beta_0, gamma_1,
           beta_1, Wp1, bp1, Wp2, bp2):
    pad = NW * EPT - E
    src = jnp.concatenate([edge_index[0], jnp.zeros((pad,), jnp.int32)])
    dst = jnp.concatenate([edge_index[1], jnp.full((pad,), N, jnp.int32)])
    srcp = src.reshape(NW, NCHUNK, CHUNK)
    dstp = dst.reshape(NW, NCHUNK, CHUNK)
    idxp = jnp.stack([srcp, dstp], axis=2).reshape(NW, NPAIR, 2, 2, CHUNK)

    h = _pre(x, W_pre, b_pre.reshape(1, H))

    layers = [(W1_0, b1_0, W2_0, b2_0), (W1_1, b1_1, W2_1, b2_1),
              (W1_2, b1_2, W2_2, b2_2)]
    bns = [(gamma_0, beta_0), (gamma_1, beta_1)]

    for l, (W1, b1, W2, b2) in enumerate(layers):
        parts = _seg_sum(h, idxp)
        if l < 2:
            u, st = _conv_stats(h, parts, W1, b1.reshape(1, H), W2,
                                b2.reshape(1, H))
            g, bb = bns[l]
            h = _bn(u, st, g.reshape(1, H), bb.reshape(1, H))
        else:
            out = _last(h, parts, W1, b1.reshape(1, H), W2, b2.reshape(1, H),
                        Wp1, bp1.reshape(1, H), Wp2, bp2.reshape(1, OUT))
    return out.reshape(1, -1)
